# baseline (speedup 1.0000x reference)
import dataclasses
import functools

import jax
import jax.numpy as jnp
from jax import lax
from jax.experimental import pallas as pl
from jax.experimental.pallas import tpu as pltpu
from jax.experimental.pallas import tpu_sc as plsc

N = 10000
E = 320000
D = 128

NW = 32
R = 320
NP = NW * R
RJ = R + 16
C1 = 1280
NCH = E // C1
BLK = 128
G = 256
CB = 400
NB = 3

NEG_INF = float("-inf")


def _proj(x_p, W, b2):
    BN = 1024

    def body(x_ref, w_ref, b_ref, a_ref, bm_ref):
        w1 = w_ref[:D, :]
        w2 = w_ref[D:, :]
        xv = x_ref[...]
        a_ref[...] = jnp.dot(
            xv, w2, preferred_element_type=jnp.float32
        ).astype(jnp.bfloat16)
        bm_ref[...] = (
            jnp.dot(xv, w1 - w2, preferred_element_type=jnp.float32) + b_ref[...]
        )

    return pl.pallas_call(
        body,
        grid=(NP // BN,),
        in_specs=[
            pl.BlockSpec((BN, D), lambda i: (i, 0)),
            pl.BlockSpec((2 * D, D), lambda i: (0, 0)),
            pl.BlockSpec((1, D), lambda i: (0, 0)),
        ],
        out_specs=[
            pl.BlockSpec((BN, D), lambda i: (i, 0)),
            pl.BlockSpec((BN, D), lambda i: (i, 0)),
        ],
        out_shape=[
            jax.ShapeDtypeStruct((NP, D), jnp.bfloat16),
            jax.ShapeDtypeStruct((NP, D), jnp.float32),
        ],
    )(x_p, W, b2)


def _post(sgm, Bmb):
    BN = 1024

    def body(s_ref, bm_ref, o_ref):
        o_ref[...] = jnp.maximum(
            s_ref[...].astype(jnp.float32) + bm_ref[...], 0.0
        )

    return pl.pallas_call(
        body,
        grid=(NP // BN,),
        in_specs=[
            pl.BlockSpec((BN, D), lambda i: (i, 0)),
            pl.BlockSpec((BN, D), lambda i: (i, 0)),
        ],
        out_specs=pl.BlockSpec((BN, D), lambda i: (i, 0)),
        out_shape=jax.ShapeDtypeStruct((NP, D), jnp.float32),
    )(sgm, Bmb)


def _segmax(A, src, dst):
    H = D // 2
    mesh = plsc.VectorSubcoreMesh(
        core_axis_name="c", subcore_axis_name="s", num_cores=2, num_subcores=16
    )
    cp = pltpu.CompilerParams()
    if "needs_layout_passes" in pltpu.CompilerParams.__dataclass_fields__:
        cp = dataclasses.replace(cp, needs_layout_passes=False)
    if "use_tc_tiling_on_sc" in pltpu.CompilerParams.__dataclass_fields__:
        cp = dataclasses.replace(cp, use_tc_tiling_on_sc=False)

    @functools.partial(
        pl.kernel,
        compiler_params=cp,
        out_type=jax.ShapeDtypeStruct((NP, D // 2), jnp.int32),
        mesh=mesh,
        scratch_types=[
            pltpu.VMEM((RJ, D // 2), jnp.int32),
            pltpu.VMEM((RJ, D // 2), jnp.int32),
            pltpu.VMEM((C1,), jnp.int32),
            pltpu.VMEM((C1,), jnp.int32),
            pltpu.VMEM((C1,), jnp.int32),
            pltpu.VMEM((C1,), jnp.int32),
            pltpu.VMEM((CB,), jnp.int32),
            pltpu.VMEM((CB,), jnp.int32),
            [pltpu.VMEM((G,), jnp.int32)] * NB,
            [pltpu.VMEM((G + 16,), jnp.int32)] * NB,
            [pltpu.VMEM((G, D // 2), jnp.int32)] * NB,
            pltpu.SemaphoreType.DMA,
            pltpu.SemaphoreType.DMA,
            [pltpu.SemaphoreType.DMA] * NB,
        ],
    )
    def k(a_hbm, src_hbm, dst_hbm, out_hbm,
          acc, acc2, srcc0, dstc0, srcc1, dstc1, csrc, cdl,
          gsrcs, gdls, rowss, csem0, csem1, gsems):
        wid = lax.axis_index("s") * 2 + lax.axis_index("c")
        lo = wid * R

        ninf_pair = plsc.bitcast(jnp.full((32,), NEG_INF, jnp.bfloat16), jnp.int32)

        @pl.loop(0, RJ)
        def _(r):
            for c in range(H // 16):
                acc[r, pl.ds(c * 16, 16)] = ninf_pair
                acc2[r, pl.ds(c * 16, 16)] = ninf_pair

        @pl.loop(0, CB, step=16)
        def _(i):
            csrc[pl.ds(i, 16)] = jnp.zeros((16,), jnp.int32)
            cdl[pl.ds(i, 16)] = jnp.full((16,), R, jnp.int32)

        gbufs = tuple(zip(gsrcs, gdls, rowss, gsems))

        def snapshot_and_issue(bi):
            gsrc, gdl, rows, gsem = gbufs[bi]

            @pl.loop(0, G, step=16)
            def _(i):
                gsrc[pl.ds(i, 16)] = csrc[pl.ds(i, 16)]
                gdl[pl.ds(i, 16)] = cdl[pl.ds(i, 16)]

            pltpu.async_copy(a_hbm.at[gsrc], rows, gsem)
            for i in range(BLK // 16):
                t = csrc[pl.ds(G + i * 16, 16)]
                csrc[pl.ds(i * 16, 16)] = t
                t2 = cdl[pl.ds(G + i * 16, 16)]
                cdl[pl.ds(i * 16, 16)] = t2

        def wait_and_accum(bi, nvalid=None):
            gsrc, gdl, rows, gsem = gbufs[bi]
            pltpu.make_async_copy(a_hbm.at[gsrc], rows, gsem).wait()

            def rmw(a_ref, d, j):
                for c in range(H // 16):
                    sl = pl.ds(c * 16, 16)
                    av = plsc.bitcast(a_ref[d, sl], jnp.bfloat16)
                    rv = plsc.bitcast(rows[j, sl], jnp.bfloat16)
                    a_ref[d, sl] = plsc.bitcast(jnp.maximum(av, rv), jnp.int32)

            if nvalid is None:
                def batch(b, _):
                    jb = b * 16
                    dvec = gdl[pl.ds(jb, 16)]
                    dscal = [dvec[e] for e in range(16)]
                    for e in range(16):
                        rmw(acc if e % 2 == 0 else acc2, dscal[e], jb + e)
                    return 0

                lax.fori_loop(0, G // 16, batch, 0)
            else:
                def body(j, _):
                    rmw(acc, gdl[pl.ds(j, 16)][0], j)
                    return 0

                lax.fori_loop(0, nvalid, body, 0)

        def flush(pb, nf):
            for bi in range(NB):
                @pl.when(pb == bi)
                def _():
                    snapshot_and_issue(bi)

                    @pl.when(nf >= NB - 1)
                    def _():
                        wait_and_accum((bi + 1) % NB)

        cbufs = ((srcc0, dstc0, csem0), (srcc1, dstc1, csem1))

        def issue_chunk(ci, bi):
            sc_, dc_, sem = cbufs[bi]
            pltpu.async_copy(src_hbm.at[pl.ds(ci * C1, C1)], sc_, sem)
            pltpu.async_copy(dst_hbm.at[pl.ds(ci * C1, C1)], dc_, sem)

        def wait_chunk(ci, bi):
            sc_, dc_, sem = cbufs[bi]
            pltpu.make_async_copy(src_hbm.at[pl.ds(ci * C1, C1)], sc_, sem).wait()
            pltpu.make_async_copy(dst_hbm.at[pl.ds(ci * C1, C1)], dc_, sem).wait()

        def process_chunk(bi, state):
            sc_, dc_, _ = cbufs[bi]

            def blk_body(blk, state):
                wp, pb, nf = state
                base = blk * BLK
                items = []
                cnts = []
                for g in range(BLK // 16):
                    o = base + g * 16
                    dv = dc_[pl.ds(o, 16)]
                    sv = sc_[pl.ds(o, 16)]
                    dl = dv - lo
                    m = plsc.bitcast(dl, jnp.uint32) < jnp.uint32(R)
                    items.append((sv, dl, m))
                    cnts.append(plsc.all_reduce_population_count(m)[0])
                offs = [wp]
                for g in range(1, BLK // 16):
                    offs.append(offs[-1] + cnts[g - 1])
                for (sv, dl, m), off in zip(items, offs):
                    plsc.store_compressed(csrc.at[pl.ds(off, 16)], sv, mask=m)
                    plsc.store_compressed(cdl.at[pl.ds(off, 16)], dl, mask=m)
                wp = offs[-1] + cnts[-1]

                full = wp >= G

                @pl.when(full)
                def _():
                    flush(pb, nf)

                wp = jnp.where(full, wp - G, wp)
                pbn = pb + 1
                pb = jnp.where(full, jnp.where(pbn == NB, 0, pbn), pb)
                nf = jnp.where(full, nf + 1, nf)
                return (wp, pb, nf)

            return lax.fori_loop(0, C1 // BLK, blk_body, state)

        def outer(i, state):
            i2 = i * 2
            issue_chunk(i2 + 1, 1)
            wait_chunk(i2, 0)
            state = process_chunk(0, state)

            @pl.when(i2 + 2 < NCH)
            def _():
                issue_chunk(i2 + 2, 0)

            wait_chunk(i2 + 1, 1)
            state = process_chunk(1, state)
            return state

        issue_chunk(0, 0)
        wp_fin, pb_fin, nf_fin = lax.fori_loop(
            0, NCH // 2, outer, (jnp.int32(0), jnp.int32(0), jnp.int32(0))
        )

        for k_back in range(NB - 1, 0, -1):
            for bi in range(NB):
                @pl.when((nf_fin >= k_back) & ((nf_fin - k_back) % NB == bi))
                def _():
                    wait_and_accum(bi)

        @pl.when(wp_fin > 0)
        def _():
            snapshot_and_issue(0)
            wait_and_accum(0, wp_fin)

        @pl.loop(0, R)
        def _(r):
            for c in range(H // 16):
                sl = pl.ds(c * 16, 16)
                av = plsc.bitcast(acc[r, sl], jnp.bfloat16)
                bv = plsc.bitcast(acc2[r, sl], jnp.bfloat16)
                acc[r, sl] = plsc.bitcast(jnp.maximum(av, bv), jnp.int32)

        pltpu.sync_copy(acc.at[pl.ds(0, R)], out_hbm.at[pl.ds(lo, R)])

    return k(A, src, dst)


def kernel(x, edge_index, W, b):
    x_p = jnp.pad(x, ((0, NP - N), (0, 0)))
    A, Bmb = _proj(x_p, W, b.reshape(1, D))
    a_i32 = jax.lax.bitcast_convert_type(A.reshape(NP, D // 2, 2), jnp.int32)
    sgm_i32 = _segmax(a_i32, edge_index[0], edge_index[1])
    sgm = jax.lax.bitcast_convert_type(sgm_i32, jnp.bfloat16).reshape(NP, D)
    out_p = _post(sgm, Bmb)
    return out_p[:N]


# speedup vs baseline: 1.0046x; 1.0046x over previous
import dataclasses
import functools

import jax
import jax.numpy as jnp
from jax import lax
from jax.experimental import pallas as pl
from jax.experimental.pallas import tpu as pltpu
from jax.experimental.pallas import tpu_sc as plsc

N = 10000
E = 320000
D = 128

NW = 32
R = 320
NP = NW * R
RJ = R + 16
C1 = 1280
NCH = E // C1
BLK = 128
G = 256
CB = 400
NB = 3

NEG_INF = float("-inf")


def _proj(x_p, W, b2):
    BN = 1024

    def body(x_ref, w_ref, b_ref, a_ref, bm_ref):
        w1 = w_ref[:D, :]
        w2 = w_ref[D:, :]
        xv = x_ref[...]
        a_ref[...] = jnp.dot(
            xv, w2, preferred_element_type=jnp.float32
        ).astype(jnp.bfloat16)
        bm_ref[...] = (
            jnp.dot(xv, w1 - w2, preferred_element_type=jnp.float32) + b_ref[...]
        )

    return pl.pallas_call(
        body,
        grid=(NP // BN,),
        in_specs=[
            pl.BlockSpec((BN, D), lambda i: (i, 0)),
            pl.BlockSpec((2 * D, D), lambda i: (0, 0)),
            pl.BlockSpec((1, D), lambda i: (0, 0)),
        ],
        out_specs=[
            pl.BlockSpec((BN, D), lambda i: (i, 0)),
            pl.BlockSpec((BN, D), lambda i: (i, 0)),
        ],
        out_shape=[
            jax.ShapeDtypeStruct((NP, D), jnp.bfloat16),
            jax.ShapeDtypeStruct((NP, D), jnp.float32),
        ],
    )(x_p, W, b2)


def _post(sgm, Bmb):
    BN = 1024

    def body(s_ref, bm_ref, o_ref):
        o_ref[...] = jnp.maximum(
            s_ref[...].astype(jnp.float32) + bm_ref[...], 0.0
        )

    return pl.pallas_call(
        body,
        grid=(NP // BN,),
        in_specs=[
            pl.BlockSpec((BN, D), lambda i: (i, 0)),
            pl.BlockSpec((BN, D), lambda i: (i, 0)),
        ],
        out_specs=pl.BlockSpec((BN, D), lambda i: (i, 0)),
        out_shape=jax.ShapeDtypeStruct((NP, D), jnp.float32),
    )(sgm, Bmb)


def _segmax(A, src, dst):
    H = D // 2
    mesh = plsc.VectorSubcoreMesh(
        core_axis_name="c", subcore_axis_name="s", num_cores=2, num_subcores=16
    )
    cp = pltpu.CompilerParams()
    if "needs_layout_passes" in pltpu.CompilerParams.__dataclass_fields__:
        cp = dataclasses.replace(cp, needs_layout_passes=False)
    if "use_tc_tiling_on_sc" in pltpu.CompilerParams.__dataclass_fields__:
        cp = dataclasses.replace(cp, use_tc_tiling_on_sc=False)

    @functools.partial(
        pl.kernel,
        compiler_params=cp,
        out_type=jax.ShapeDtypeStruct((NP, D // 2), jnp.int32),
        mesh=mesh,
        scratch_types=[
            pltpu.VMEM((RJ, D // 2), jnp.int32),
            pltpu.VMEM((C1,), jnp.int32),
            pltpu.VMEM((C1,), jnp.int32),
            pltpu.VMEM((C1,), jnp.int32),
            pltpu.VMEM((C1,), jnp.int32),
            pltpu.VMEM((CB,), jnp.int32),
            pltpu.VMEM((CB,), jnp.int32),
            [pltpu.VMEM((G,), jnp.int32)] * NB,
            [pltpu.VMEM((G + 16,), jnp.int32)] * NB,
            [pltpu.VMEM((G, D // 2), jnp.int32)] * NB,
            pltpu.SemaphoreType.DMA,
            pltpu.SemaphoreType.DMA,
            [pltpu.SemaphoreType.DMA] * NB,
        ],
    )
    def k(a_hbm, src_hbm, dst_hbm, out_hbm,
          acc, srcc0, dstc0, srcc1, dstc1, csrc, cdl,
          gsrcs, gdls, rowss, csem0, csem1, gsems):
        wid = lax.axis_index("s") * 2 + lax.axis_index("c")
        lo = wid * R

        ninf_pair = plsc.bitcast(jnp.full((32,), NEG_INF, jnp.bfloat16), jnp.int32)

        @pl.loop(0, RJ)
        def _(r):
            for c in range(H // 16):
                acc[r, pl.ds(c * 16, 16)] = ninf_pair

        @pl.loop(0, CB, step=16)
        def _(i):
            csrc[pl.ds(i, 16)] = jnp.zeros((16,), jnp.int32)
            cdl[pl.ds(i, 16)] = jnp.full((16,), R, jnp.int32)

        gbufs = tuple(zip(gsrcs, gdls, rowss, gsems))

        def snapshot_and_issue(bi):
            gsrc, gdl, rows, gsem = gbufs[bi]

            @pl.loop(0, G, step=16)
            def _(i):
                gsrc[pl.ds(i, 16)] = csrc[pl.ds(i, 16)]
                gdl[pl.ds(i, 16)] = cdl[pl.ds(i, 16)]

            pltpu.async_copy(a_hbm.at[gsrc], rows, gsem)
            for i in range(BLK // 16):
                t = csrc[pl.ds(G + i * 16, 16)]
                csrc[pl.ds(i * 16, 16)] = t
                t2 = cdl[pl.ds(G + i * 16, 16)]
                cdl[pl.ds(i * 16, 16)] = t2

        def wait_and_accum(bi, nvalid=None):
            gsrc, gdl, rows, gsem = gbufs[bi]
            pltpu.make_async_copy(a_hbm.at[gsrc], rows, gsem).wait()

            def rmw(d, j):
                for c in range(H // 16):
                    sl = pl.ds(c * 16, 16)
                    av = plsc.bitcast(acc[d, sl], jnp.bfloat16)
                    rv = plsc.bitcast(rows[j, sl], jnp.bfloat16)
                    acc[d, sl] = plsc.bitcast(jnp.maximum(av, rv), jnp.int32)

            if nvalid is None:
                def batch(b, _):
                    jb = b * 16
                    dvec = gdl[pl.ds(jb, 16)]
                    dscal = [dvec[e] for e in range(16)]
                    for e in range(16):
                        rmw(dscal[e], jb + e)
                    return 0

                lax.fori_loop(0, G // 16, batch, 0)
            else:
                def body(j, _):
                    rmw(gdl[pl.ds(j, 16)][0], j)
                    return 0

                lax.fori_loop(0, nvalid, body, 0)

        def flush(pb, nf):
            for bi in range(NB):
                @pl.when(pb == bi)
                def _():
                    snapshot_and_issue(bi)

                    @pl.when(nf >= NB - 1)
                    def _():
                        wait_and_accum((bi + 1) % NB)

        cbufs = ((srcc0, dstc0, csem0), (srcc1, dstc1, csem1))

        def issue_chunk(ci, bi):
            sc_, dc_, sem = cbufs[bi]
            pltpu.async_copy(src_hbm.at[pl.ds(ci * C1, C1)], sc_, sem)
            pltpu.async_copy(dst_hbm.at[pl.ds(ci * C1, C1)], dc_, sem)

        def wait_chunk(ci, bi):
            sc_, dc_, sem = cbufs[bi]
            pltpu.make_async_copy(src_hbm.at[pl.ds(ci * C1, C1)], sc_, sem).wait()
            pltpu.make_async_copy(dst_hbm.at[pl.ds(ci * C1, C1)], dc_, sem).wait()

        def process_chunk(bi, state):
            sc_, dc_, _ = cbufs[bi]

            def blk_body(blk, state):
                wp, pb, nf = state
                base = blk * BLK
                items = []
                cnts = []
                for g in range(BLK // 16):
                    o = base + g * 16
                    dv = dc_[pl.ds(o, 16)]
                    sv = sc_[pl.ds(o, 16)]
                    dl = dv - lo
                    m = plsc.bitcast(dl, jnp.uint32) < jnp.uint32(R)
                    items.append((sv, dl, m))
                    cnts.append(plsc.all_reduce_population_count(m)[0])
                offs = [wp]
                for g in range(1, BLK // 16):
                    offs.append(offs[-1] + cnts[g - 1])
                for (sv, dl, m), off in zip(items, offs):
                    plsc.store_compressed(csrc.at[pl.ds(off, 16)], sv, mask=m)
                    plsc.store_compressed(cdl.at[pl.ds(off, 16)], dl, mask=m)
                wp = offs[-1] + cnts[-1]

                full = wp >= G

                @pl.when(full)
                def _():
                    flush(pb, nf)

                wp = jnp.where(full, wp - G, wp)
                pbn = pb + 1
                pb = jnp.where(full, jnp.where(pbn == NB, 0, pbn), pb)
                nf = jnp.where(full, nf + 1, nf)
                return (wp, pb, nf)

            return lax.fori_loop(0, C1 // BLK, blk_body, state)

        def outer(i, state):
            i2 = i * 2
            issue_chunk(i2 + 1, 1)
            wait_chunk(i2, 0)
            state = process_chunk(0, state)

            @pl.when(i2 + 2 < NCH)
            def _():
                issue_chunk(i2 + 2, 0)

            wait_chunk(i2 + 1, 1)
            state = process_chunk(1, state)
            return state

        issue_chunk(0, 0)
        wp_fin, pb_fin, nf_fin = lax.fori_loop(
            0, NCH // 2, outer, (jnp.int32(0), jnp.int32(0), jnp.int32(0))
        )

        for k_back in range(NB - 1, 0, -1):
            for bi in range(NB):
                @pl.when((nf_fin >= k_back) & ((nf_fin - k_back) % NB == bi))
                def _():
                    wait_and_accum(bi)

        @pl.when(wp_fin > 0)
        def _():
            snapshot_and_issue(0)
            wait_and_accum(0, wp_fin)

        pltpu.sync_copy(acc.at[pl.ds(0, R)], out_hbm.at[pl.ds(lo, R)])

    return k(A, src, dst)


def kernel(x, edge_index, W, b):
    x_p = jnp.pad(x, ((0, NP - N), (0, 0)))
    A, Bmb = _proj(x_p, W, b.reshape(1, D))
    a_i32 = jax.lax.bitcast_convert_type(A.reshape(NP, D // 2, 2), jnp.int32)
    sgm_i32 = _segmax(a_i32, edge_index[0], edge_index[1])
    sgm = jax.lax.bitcast_convert_type(sgm_i32, jnp.bfloat16).reshape(NP, D)
    out_p = _post(sgm, Bmb)
    return out_p[:N]


# speedup vs baseline: 1.0961x; 1.0910x over previous
import dataclasses
import functools

import jax
import jax.numpy as jnp
from jax import lax
from jax.experimental import pallas as pl
from jax.experimental.pallas import tpu as pltpu
from jax.experimental.pallas import tpu_sc as plsc

N = 10000
E = 320000
D = 128

NW = 32
R = 320
NP = NW * R
RJ = R + 16
C1 = 3200
NCH = E // C1
BLK = 128
G = 256
CB = 400
NB = 3

NEG_INF = float("-inf")


def _proj(x_p, W, b2):
    BN = 1024

    def body(x_ref, w_ref, b_ref, a_ref, bm_ref):
        w1 = w_ref[:D, :]
        w2 = w_ref[D:, :]
        xv = x_ref[...]
        a_ref[...] = jnp.dot(
            xv, w2, preferred_element_type=jnp.float32
        ).astype(jnp.bfloat16)
        bm_ref[...] = (
            jnp.dot(xv, w1 - w2, preferred_element_type=jnp.float32) + b_ref[...]
        )

    return pl.pallas_call(
        body,
        grid=(NP // BN,),
        in_specs=[
            pl.BlockSpec((BN, D), lambda i: (i, 0)),
            pl.BlockSpec((2 * D, D), lambda i: (0, 0)),
            pl.BlockSpec((1, D), lambda i: (0, 0)),
        ],
        out_specs=[
            pl.BlockSpec((BN, D), lambda i: (i, 0)),
            pl.BlockSpec((BN, D), lambda i: (i, 0)),
        ],
        out_shape=[
            jax.ShapeDtypeStruct((NP, D), jnp.bfloat16),
            jax.ShapeDtypeStruct((NP, D), jnp.float32),
        ],
    )(x_p, W, b2)


def _post(sgm, Bmb):
    BN = 1024

    def body(s_ref, bm_ref, o_ref):
        o_ref[...] = jnp.maximum(
            s_ref[...].astype(jnp.float32) + bm_ref[...], 0.0
        )

    return pl.pallas_call(
        body,
        grid=(NP // BN,),
        in_specs=[
            pl.BlockSpec((BN, D), lambda i: (i, 0)),
            pl.BlockSpec((BN, D), lambda i: (i, 0)),
        ],
        out_specs=pl.BlockSpec((BN, D), lambda i: (i, 0)),
        out_shape=jax.ShapeDtypeStruct((NP, D), jnp.float32),
    )(sgm, Bmb)


def _segmax(A, src, dst):
    H = D // 2
    mesh = plsc.VectorSubcoreMesh(
        core_axis_name="c", subcore_axis_name="s", num_cores=2, num_subcores=16
    )
    cp = pltpu.CompilerParams()
    if "needs_layout_passes" in pltpu.CompilerParams.__dataclass_fields__:
        cp = dataclasses.replace(cp, needs_layout_passes=False)
    if "use_tc_tiling_on_sc" in pltpu.CompilerParams.__dataclass_fields__:
        cp = dataclasses.replace(cp, use_tc_tiling_on_sc=False)

    @functools.partial(
        pl.kernel,
        compiler_params=cp,
        out_type=jax.ShapeDtypeStruct((NP, D // 2), jnp.int32),
        mesh=mesh,
        scratch_types=[
            pltpu.VMEM((RJ, D // 2), jnp.int32),
            pltpu.VMEM((C1,), jnp.int32),
            pltpu.VMEM((C1,), jnp.int32),
            pltpu.VMEM((C1,), jnp.int32),
            pltpu.VMEM((C1,), jnp.int32),
            pltpu.VMEM((CB,), jnp.int32),
            pltpu.VMEM((CB,), jnp.int32),
            [pltpu.VMEM((G,), jnp.int32)] * NB,
            [pltpu.VMEM((G + 16,), jnp.int32)] * NB,
            [pltpu.VMEM((G, D // 2), jnp.int32)] * NB,
            pltpu.SemaphoreType.DMA,
            pltpu.SemaphoreType.DMA,
            [pltpu.SemaphoreType.DMA] * NB,
        ],
    )
    def k(a_hbm, src_hbm, dst_hbm, out_hbm,
          acc, srcc0, dstc0, srcc1, dstc1, csrc, cdl,
          gsrcs, gdls, rowss, csem0, csem1, gsems):
        wid = lax.axis_index("s") * 2 + lax.axis_index("c")
        lo = wid * R

        ninf_pair = plsc.bitcast(jnp.full((32,), NEG_INF, jnp.bfloat16), jnp.int32)

        @pl.loop(0, RJ)
        def _(r):
            for c in range(H // 16):
                acc[r, pl.ds(c * 16, 16)] = ninf_pair

        @pl.loop(0, CB, step=16)
        def _(i):
            csrc[pl.ds(i, 16)] = jnp.zeros((16,), jnp.int32)
            cdl[pl.ds(i, 16)] = jnp.full((16,), R, jnp.int32)

        gbufs = tuple(zip(gsrcs, gdls, rowss, gsems))

        def snapshot_and_issue(bi):
            gsrc, gdl, rows, gsem = gbufs[bi]

            @pl.loop(0, G, step=16)
            def _(i):
                gsrc[pl.ds(i, 16)] = csrc[pl.ds(i, 16)]
                gdl[pl.ds(i, 16)] = cdl[pl.ds(i, 16)]

            pltpu.async_copy(a_hbm.at[gsrc], rows, gsem)
            for i in range(BLK // 16):
                t = csrc[pl.ds(G + i * 16, 16)]
                csrc[pl.ds(i * 16, 16)] = t
                t2 = cdl[pl.ds(G + i * 16, 16)]
                cdl[pl.ds(i * 16, 16)] = t2

        def wait_and_accum(bi, nvalid=None):
            gsrc, gdl, rows, gsem = gbufs[bi]
            pltpu.make_async_copy(a_hbm.at[gsrc], rows, gsem).wait()

            def rmw(d, j):
                for c in range(H // 16):
                    sl = pl.ds(c * 16, 16)
                    av = plsc.bitcast(acc[d, sl], jnp.bfloat16)
                    rv = plsc.bitcast(rows[j, sl], jnp.bfloat16)
                    acc[d, sl] = plsc.bitcast(jnp.maximum(av, rv), jnp.int32)

            if nvalid is None:
                def batch(b, _):
                    jb = b * 16
                    dvec = gdl[pl.ds(jb, 16)]
                    dscal = [dvec[e] for e in range(16)]
                    for e in range(16):
                        rmw(dscal[e], jb + e)
                    return 0

                lax.fori_loop(0, G // 16, batch, 0)
            else:
                def body(j, _):
                    rmw(gdl[pl.ds(j, 16)][0], j)
                    return 0

                lax.fori_loop(0, nvalid, body, 0)

        def flush(pb, nf):
            for bi in range(NB):
                @pl.when(pb == bi)
                def _():
                    snapshot_and_issue(bi)

                    @pl.when(nf >= NB - 1)
                    def _():
                        wait_and_accum((bi + 1) % NB)

        cbufs = ((srcc0, dstc0, csem0), (srcc1, dstc1, csem1))

        def issue_chunk(ci, bi):
            sc_, dc_, sem = cbufs[bi]
            pltpu.async_copy(src_hbm.at[pl.ds(ci * C1, C1)], sc_, sem)
            pltpu.async_copy(dst_hbm.at[pl.ds(ci * C1, C1)], dc_, sem)

        def wait_chunk(ci, bi):
            sc_, dc_, sem = cbufs[bi]
            pltpu.make_async_copy(src_hbm.at[pl.ds(ci * C1, C1)], sc_, sem).wait()
            pltpu.make_async_copy(dst_hbm.at[pl.ds(ci * C1, C1)], dc_, sem).wait()

        def process_chunk(bi, state):
            sc_, dc_, _ = cbufs[bi]

            def blk_body(blk, state):
                wp, pb, nf = state
                base = blk * BLK
                items = []
                cnts = []
                for g in range(BLK // 16):
                    o = base + g * 16
                    dv = dc_[pl.ds(o, 16)]
                    sv = sc_[pl.ds(o, 16)]
                    dl = dv - lo
                    m = plsc.bitcast(dl, jnp.uint32) < jnp.uint32(R)
                    items.append((sv, dl, m))
                    cnts.append(plsc.all_reduce_population_count(m)[0])
                offs = [wp]
                for g in range(1, BLK // 16):
                    offs.append(offs[-1] + cnts[g - 1])
                for (sv, dl, m), off in zip(items, offs):
                    plsc.store_compressed(csrc.at[pl.ds(off, 16)], sv, mask=m)
                    plsc.store_compressed(cdl.at[pl.ds(off, 16)], dl, mask=m)
                wp = offs[-1] + cnts[-1]

                full = wp >= G

                @pl.when(full)
                def _():
                    flush(pb, nf)

                wp = jnp.where(full, wp - G, wp)
                pbn = pb + 1
                pb = jnp.where(full, jnp.where(pbn == NB, 0, pbn), pb)
                nf = jnp.where(full, nf + 1, nf)
                return (wp, pb, nf)

            return lax.fori_loop(0, C1 // BLK, blk_body, state)

        def outer(i, state):
            i2 = i * 2
            issue_chunk(i2 + 1, 1)
            wait_chunk(i2, 0)
            state = process_chunk(0, state)

            @pl.when(i2 + 2 < NCH)
            def _():
                issue_chunk(i2 + 2, 0)

            wait_chunk(i2 + 1, 1)
            state = process_chunk(1, state)
            return state

        issue_chunk(0, 0)
        wp_fin, pb_fin, nf_fin = lax.fori_loop(
            0, NCH // 2, outer, (jnp.int32(0), jnp.int32(0), jnp.int32(0))
        )

        for k_back in range(NB - 1, 0, -1):
            for bi in range(NB):
                @pl.when((nf_fin >= k_back) & ((nf_fin - k_back) % NB == bi))
                def _():
                    wait_and_accum(bi)

        @pl.when(wp_fin > 0)
        def _():
            snapshot_and_issue(0)
            wait_and_accum(0, wp_fin)

        pltpu.sync_copy(acc.at[pl.ds(0, R)], out_hbm.at[pl.ds(lo, R)])

    return k(A, src, dst)


def kernel(x, edge_index, W, b):
    x_p = jnp.pad(x, ((0, NP - N), (0, 0)))
    A, Bmb = _proj(x_p, W, b.reshape(1, D))
    a_i32 = jax.lax.bitcast_convert_type(A.reshape(NP, D // 2, 2), jnp.int32)
    sgm_i32 = _segmax(a_i32, edge_index[0], edge_index[1])
    sgm = jax.lax.bitcast_convert_type(sgm_i32, jnp.bfloat16).reshape(NP, D)
    out_p = _post(sgm, Bmb)
    return out_p[:N]


# speedup vs baseline: 1.1049x; 1.0080x over previous
import dataclasses
import functools

import jax
import jax.numpy as jnp
from jax import lax
from jax.experimental import pallas as pl
from jax.experimental.pallas import tpu as pltpu
from jax.experimental.pallas import tpu_sc as plsc

N = 10000
E = 320000
D = 128

NW = 32
R = 320
NP = NW * R
RJ = R + 16
C1 = 6400
NCH = E // C1
BLK = 128
G = 256
CB = 400
NB = 3

NEG_INF = float("-inf")


def _proj(x_p, W, b2):
    BN = 1024

    def body(x_ref, w_ref, b_ref, a_ref, bm_ref):
        w1 = w_ref[:D, :]
        w2 = w_ref[D:, :]
        xv = x_ref[...]
        a_ref[...] = jnp.dot(
            xv, w2, preferred_element_type=jnp.float32
        ).astype(jnp.bfloat16)
        bm_ref[...] = (
            jnp.dot(xv, w1 - w2, preferred_element_type=jnp.float32) + b_ref[...]
        )

    return pl.pallas_call(
        body,
        grid=(NP // BN,),
        in_specs=[
            pl.BlockSpec((BN, D), lambda i: (i, 0)),
            pl.BlockSpec((2 * D, D), lambda i: (0, 0)),
            pl.BlockSpec((1, D), lambda i: (0, 0)),
        ],
        out_specs=[
            pl.BlockSpec((BN, D), lambda i: (i, 0)),
            pl.BlockSpec((BN, D), lambda i: (i, 0)),
        ],
        out_shape=[
            jax.ShapeDtypeStruct((NP, D), jnp.bfloat16),
            jax.ShapeDtypeStruct((NP, D), jnp.float32),
        ],
    )(x_p, W, b2)


def _post(sgm, Bmb):
    BN = 1024

    def body(s_ref, bm_ref, o_ref):
        o_ref[...] = jnp.maximum(
            s_ref[...].astype(jnp.float32) + bm_ref[...], 0.0
        )

    return pl.pallas_call(
        body,
        grid=(NP // BN,),
        in_specs=[
            pl.BlockSpec((BN, D), lambda i: (i, 0)),
            pl.BlockSpec((BN, D), lambda i: (i, 0)),
        ],
        out_specs=pl.BlockSpec((BN, D), lambda i: (i, 0)),
        out_shape=jax.ShapeDtypeStruct((NP, D), jnp.float32),
    )(sgm, Bmb)


def _segmax(A, src, dst):
    H = D // 2
    mesh = plsc.VectorSubcoreMesh(
        core_axis_name="c", subcore_axis_name="s", num_cores=2, num_subcores=16
    )
    cp = pltpu.CompilerParams()
    if "needs_layout_passes" in pltpu.CompilerParams.__dataclass_fields__:
        cp = dataclasses.replace(cp, needs_layout_passes=False)
    if "use_tc_tiling_on_sc" in pltpu.CompilerParams.__dataclass_fields__:
        cp = dataclasses.replace(cp, use_tc_tiling_on_sc=False)

    @functools.partial(
        pl.kernel,
        compiler_params=cp,
        out_type=jax.ShapeDtypeStruct((NP, D // 2), jnp.int32),
        mesh=mesh,
        scratch_types=[
            pltpu.VMEM((RJ, D // 2), jnp.int32),
            pltpu.VMEM((C1,), jnp.int32),
            pltpu.VMEM((C1,), jnp.int32),
            pltpu.VMEM((C1,), jnp.int32),
            pltpu.VMEM((C1,), jnp.int32),
            pltpu.VMEM((CB,), jnp.int32),
            pltpu.VMEM((CB,), jnp.int32),
            [pltpu.VMEM((G,), jnp.int32)] * NB,
            [pltpu.VMEM((G + 16,), jnp.int32)] * NB,
            [pltpu.VMEM((G, D // 2), jnp.int32)] * NB,
            pltpu.SemaphoreType.DMA,
            pltpu.SemaphoreType.DMA,
            [pltpu.SemaphoreType.DMA] * NB,
        ],
    )
    def k(a_hbm, src_hbm, dst_hbm, out_hbm,
          acc, srcc0, dstc0, srcc1, dstc1, csrc, cdl,
          gsrcs, gdls, rowss, csem0, csem1, gsems):
        wid = lax.axis_index("s") * 2 + lax.axis_index("c")
        lo = wid * R

        ninf_pair = plsc.bitcast(jnp.full((32,), NEG_INF, jnp.bfloat16), jnp.int32)

        @pl.loop(0, RJ)
        def _(r):
            for c in range(H // 16):
                acc[r, pl.ds(c * 16, 16)] = ninf_pair

        @pl.loop(0, CB, step=16)
        def _(i):
            csrc[pl.ds(i, 16)] = jnp.zeros((16,), jnp.int32)
            cdl[pl.ds(i, 16)] = jnp.full((16,), R, jnp.int32)

        gbufs = tuple(zip(gsrcs, gdls, rowss, gsems))

        def snapshot_and_issue(bi):
            gsrc, gdl, rows, gsem = gbufs[bi]

            @pl.loop(0, G, step=16)
            def _(i):
                gsrc[pl.ds(i, 16)] = csrc[pl.ds(i, 16)]
                gdl[pl.ds(i, 16)] = cdl[pl.ds(i, 16)]

            pltpu.async_copy(a_hbm.at[gsrc], rows, gsem)
            for i in range(BLK // 16):
                t = csrc[pl.ds(G + i * 16, 16)]
                csrc[pl.ds(i * 16, 16)] = t
                t2 = cdl[pl.ds(G + i * 16, 16)]
                cdl[pl.ds(i * 16, 16)] = t2

        def wait_and_accum(bi, nvalid=None):
            gsrc, gdl, rows, gsem = gbufs[bi]
            pltpu.make_async_copy(a_hbm.at[gsrc], rows, gsem).wait()

            def rmw(d, j):
                for c in range(H // 16):
                    sl = pl.ds(c * 16, 16)
                    av = plsc.bitcast(acc[d, sl], jnp.bfloat16)
                    rv = plsc.bitcast(rows[j, sl], jnp.bfloat16)
                    acc[d, sl] = plsc.bitcast(jnp.maximum(av, rv), jnp.int32)

            if nvalid is None:
                def batch(b, _):
                    jb = b * 16
                    dvec = gdl[pl.ds(jb, 16)]
                    dscal = [dvec[e] for e in range(16)]
                    for e in range(16):
                        rmw(dscal[e], jb + e)
                    return 0

                lax.fori_loop(0, G // 16, batch, 0)
            else:
                def body(j, _):
                    rmw(gdl[pl.ds(j, 16)][0], j)
                    return 0

                lax.fori_loop(0, nvalid, body, 0)

        def flush(pb, nf):
            for bi in range(NB):
                @pl.when(pb == bi)
                def _():
                    snapshot_and_issue(bi)

                    @pl.when(nf >= NB - 1)
                    def _():
                        wait_and_accum((bi + 1) % NB)

        cbufs = ((srcc0, dstc0, csem0), (srcc1, dstc1, csem1))

        def issue_chunk(ci, bi):
            sc_, dc_, sem = cbufs[bi]
            pltpu.async_copy(src_hbm.at[pl.ds(ci * C1, C1)], sc_, sem)
            pltpu.async_copy(dst_hbm.at[pl.ds(ci * C1, C1)], dc_, sem)

        def wait_chunk(ci, bi):
            sc_, dc_, sem = cbufs[bi]
            pltpu.make_async_copy(src_hbm.at[pl.ds(ci * C1, C1)], sc_, sem).wait()
            pltpu.make_async_copy(dst_hbm.at[pl.ds(ci * C1, C1)], dc_, sem).wait()

        def process_chunk(bi, state):
            sc_, dc_, _ = cbufs[bi]

            def blk_body(blk, state):
                wp, pb, nf = state
                base = blk * BLK
                items = []
                cnts = []
                for g in range(BLK // 16):
                    o = base + g * 16
                    dv = dc_[pl.ds(o, 16)]
                    sv = sc_[pl.ds(o, 16)]
                    dl = dv - lo
                    m = plsc.bitcast(dl, jnp.uint32) < jnp.uint32(R)
                    items.append((sv, dl, m))
                    cnts.append(plsc.all_reduce_population_count(m)[0])
                offs = [wp]
                for g in range(1, BLK // 16):
                    offs.append(offs[-1] + cnts[g - 1])
                for (sv, dl, m), off in zip(items, offs):
                    plsc.store_compressed(csrc.at[pl.ds(off, 16)], sv, mask=m)
                    plsc.store_compressed(cdl.at[pl.ds(off, 16)], dl, mask=m)
                wp = offs[-1] + cnts[-1]

                full = wp >= G

                @pl.when(full)
                def _():
                    flush(pb, nf)

                wp = jnp.where(full, wp - G, wp)
                pbn = pb + 1
                pb = jnp.where(full, jnp.where(pbn == NB, 0, pbn), pb)
                nf = jnp.where(full, nf + 1, nf)
                return (wp, pb, nf)

            return lax.fori_loop(0, C1 // BLK, blk_body, state)

        def outer(i, state):
            i2 = i * 2
            issue_chunk(i2 + 1, 1)
            wait_chunk(i2, 0)
            state = process_chunk(0, state)

            @pl.when(i2 + 2 < NCH)
            def _():
                issue_chunk(i2 + 2, 0)

            wait_chunk(i2 + 1, 1)
            state = process_chunk(1, state)
            return state

        issue_chunk(0, 0)
        wp_fin, pb_fin, nf_fin = lax.fori_loop(
            0, NCH // 2, outer, (jnp.int32(0), jnp.int32(0), jnp.int32(0))
        )

        for k_back in range(NB - 1, 0, -1):
            for bi in range(NB):
                @pl.when((nf_fin >= k_back) & ((nf_fin - k_back) % NB == bi))
                def _():
                    wait_and_accum(bi)

        @pl.when(wp_fin > 0)
        def _():
            snapshot_and_issue(0)
            wait_and_accum(0, wp_fin)

        pltpu.sync_copy(acc.at[pl.ds(0, R)], out_hbm.at[pl.ds(lo, R)])

    return k(A, src, dst)


def kernel(x, edge_index, W, b):
    x_p = jnp.pad(x, ((0, NP - N), (0, 0)))
    A, Bmb = _proj(x_p, W, b.reshape(1, D))
    a_i32 = jax.lax.bitcast_convert_type(A.reshape(NP, D // 2, 2), jnp.int32)
    sgm_i32 = _segmax(a_i32, edge_index[0], edge_index[1])
    sgm = jax.lax.bitcast_convert_type(sgm_i32, jnp.bfloat16).reshape(NP, D)
    out_p = _post(sgm, Bmb)
    return out_p[:N]
